# Initial kernel scaffold; baseline (speedup 1.0000x reference)
import functools

import jax
import jax.numpy as jnp
from jax import lax
from jax.experimental import pallas as pl
from jax.experimental.pallas import tpu as pltpu
from jax.experimental.pallas import tpu_sc as plsc

NC = 2
NS = 16
L = 16
EPS = 1e-5


def _sc_degree(col, n):
    e = col.shape[0]
    nw = NC * NS
    ec = e // nw
    mesh = plsc.VectorSubcoreMesh(
        core_axis_name="c", subcore_axis_name="s", num_cores=NC, num_subcores=NS
    )

    @functools.partial(
        pl.kernel,
        mesh=mesh,
        out_type=jax.ShapeDtypeStruct((nw, n), jnp.float32),
        scratch_types=[
            pltpu.VMEM((ec,), jnp.int32),
            pltpu.VMEM((n,), jnp.float32),
        ],
    )
    def k(col_hbm, deg_hbm, colv, degv):
        wid = lax.axis_index("s") * NC + lax.axis_index("c")
        pltpu.sync_copy(col_hbm.at[pl.ds(wid * ec, ec)], colv)

        def zbody(i, carry):
            degv[pl.ds(i * L, L)] = jnp.zeros((L,), jnp.float32)
            return carry

        lax.fori_loop(0, n // L, zbody, 0, unroll=4)

        ones = jnp.ones((L,), jnp.float32)

        def cbody(i, carry):
            idx = colv[pl.ds(i * L, L)]
            plsc.addupdate_scatter(degv, [idx], ones)
            return carry

        lax.fori_loop(0, ec // L, cbody, 0, unroll=4)
        pltpu.sync_copy(degv, deg_hbm.at[wid])

    return k(col)


def _sc_aggregate(u, row, col, n):
    e = row.shape[0]
    c_dim = u.shape[1]
    ec = e // (NC * NS)
    K = 80
    nchunk = ec // K
    rpt = n // NS
    zr = 125
    mesh = plsc.VectorSubcoreMesh(
        core_axis_name="c", subcore_axis_name="s", num_cores=NC, num_subcores=NS
    )

    @functools.partial(
        pl.kernel,
        mesh=mesh,
        out_type=jax.ShapeDtypeStruct((NC, n, c_dim), jnp.float32),
        scratch_types=[
            pltpu.VMEM((2, K), jnp.int32),
            pltpu.VMEM((2, K), jnp.int32),
            pltpu.VMEM((2, K, c_dim), jnp.float32),
            pltpu.VMEM((125, c_dim), jnp.float32),
            pltpu.VMEM_SHARED((n, c_dim), jnp.float32),
            pltpu.SemaphoreType.DMA,
            pltpu.SemaphoreType.DMA,
        ],
    )
    def k(u_hbm, row_hbm, col_hbm, out_hbm, rowv, colv, gbuf, zbuf, acc, sem0, sem1):
        cid = lax.axis_index("c")
        sid = lax.axis_index("s")

        zr = 125
        nvec = c_dim // L

        def zbody(i, carry):
            r = i // nvec
            col0 = (i % nvec) * L
            zbuf[r, pl.ds(col0, L)] = jnp.zeros((L,), jnp.float32)
            return carry

        lax.fori_loop(0, zr * nvec, zbody, 0, unroll=4)
        for m in range(rpt // zr):
            pltpu.sync_copy(zbuf, acc.at[pl.ds(sid * rpt + m * zr, zr)])
        plsc.subcore_barrier()

        base = (cid * NS + sid) * ec
        sems = (sem0, sem1)

        def stage(j, slot):
            off = base + j * K
            pltpu.sync_copy(row_hbm.at[pl.ds(off, K)], rowv.at[slot])
            pltpu.sync_copy(col_hbm.at[pl.ds(off, K)], colv.at[slot])
            return pltpu.async_copy(u_hbm.at[rowv.at[slot]], gbuf.at[slot], sems[slot])

        def drain(slot, cp):
            cp.wait()
            pltpu.sync_copy(gbuf.at[slot], acc.at[colv.at[slot]], add=True)

        cp = stage(0, 0)
        for j in range(1, nchunk):
            cp_next = stage(j, j % 2)
            drain((j - 1) % 2, cp)
            cp = cp_next
        drain((nchunk - 1) % 2, cp)
        plsc.subcore_barrier()

        pltpu.sync_copy(
            acc.at[pl.ds(sid * rpt, rpt)], out_hbm.at[cid, pl.ds(sid * rpt, rpt)]
        )

    return k(u, row, col)


def _tc_prep(deg_part, x):
    n, c_dim = x.shape

    def k(dp_ref, x_ref, u_ref, dinv_ref):
        deg = 1.0 + jnp.sum(dp_ref[...], axis=0)
        dinv = lax.rsqrt(deg)
        u_ref[...] = x_ref[...] * dinv[:, None]
        dinv_ref[...] = dinv[:, None]

    return pl.pallas_call(
        k,
        out_shape=[
            jax.ShapeDtypeStruct((n, c_dim), jnp.float32),
            jax.ShapeDtypeStruct((n, 1), jnp.float32),
        ],
    )(deg_part, x)


def _tc_linear(agg, x, dinv, W, b):
    n, c_dim = x.shape
    blk = 1000
    grid = n // blk

    def k(agg_ref, x_ref, dinv_ref, w_ref, b_ref, y_ref, st_ref):
        i = pl.program_id(0)
        dinv = dinv_ref[...]
        pre = agg_ref[0] * dinv + agg_ref[1] * dinv + x_ref[...] * (dinv * dinv)
        y = jnp.dot(pre, w_ref[...], preferred_element_type=jnp.float32) + b_ref[...]
        y_ref[...] = y

        @pl.when(i == 0)
        def _():
            st_ref[...] = jnp.zeros_like(st_ref)

        st_ref[0:1, :] += jnp.sum(y, axis=0, keepdims=True)
        st_ref[1:2, :] += jnp.sum(y * y, axis=0, keepdims=True)

    return pl.pallas_call(
        k,
        grid=(grid,),
        in_specs=[
            pl.BlockSpec((2, blk, c_dim), lambda i: (0, i, 0)),
            pl.BlockSpec((blk, c_dim), lambda i: (i, 0)),
            pl.BlockSpec((blk, 1), lambda i: (i, 0)),
            pl.BlockSpec((c_dim, c_dim), lambda i: (0, 0)),
            pl.BlockSpec((1, c_dim), lambda i: (0, 0)),
        ],
        out_specs=[
            pl.BlockSpec((blk, c_dim), lambda i: (i, 0)),
            pl.BlockSpec((2, c_dim), lambda i: (0, 0)),
        ],
        out_shape=[
            jax.ShapeDtypeStruct((n, c_dim), jnp.float32),
            jax.ShapeDtypeStruct((2, c_dim), jnp.float32),
        ],
    )(agg, x, dinv, W, b)


def _tc_bn(y, st, gamma, beta, n):
    _, c_dim = y.shape
    blk = 1000
    grid = n // blk
    inv_n = 1.0 / n

    def k(y_ref, st_ref, g_ref, bt_ref, o_ref):
        mean = st_ref[0:1, :] * inv_n
        var = st_ref[1:2, :] * inv_n - mean * mean
        scale = lax.rsqrt(var + EPS) * g_ref[...]
        o_ref[...] = jnp.maximum((y_ref[...] - mean) * scale + bt_ref[...], 0.0)

    return pl.pallas_call(
        k,
        grid=(grid,),
        in_specs=[
            pl.BlockSpec((blk, c_dim), lambda i: (i, 0)),
            pl.BlockSpec((2, c_dim), lambda i: (0, 0)),
            pl.BlockSpec((1, c_dim), lambda i: (0, 0)),
            pl.BlockSpec((1, c_dim), lambda i: (0, 0)),
        ],
        out_specs=pl.BlockSpec((blk, c_dim), lambda i: (i, 0)),
        out_shape=jax.ShapeDtypeStruct((n, c_dim), jnp.float32),
    )(y, st, gamma, beta)


def kernel(x, edge_index, W, b, gamma, beta):
    n = x.shape[0]
    row = edge_index[0]
    col = edge_index[1]
    deg_part = _sc_degree(col, n)
    u, dinv = _tc_prep(deg_part, x)
    agg = _sc_aggregate(u, row, col, n)
    y, st = _tc_linear(agg, x, dinv, W, b.reshape(1, -1))
    return _tc_bn(y, st, gamma.reshape(1, -1), beta.reshape(1, -1), n)


# speedup vs baseline: 26.7801x; 26.7801x over previous
import functools

import jax
import jax.numpy as jnp
from jax import lax
from jax.experimental import pallas as pl
from jax.experimental.pallas import tpu as pltpu
from jax.experimental.pallas import tpu_sc as plsc

NC = 2
NS = 16
L = 16
EPS = 1e-5


def _sc_degree(col, n):
    e = col.shape[0]
    nw = NC * NS
    ec = e // nw
    mesh = plsc.VectorSubcoreMesh(
        core_axis_name="c", subcore_axis_name="s", num_cores=NC, num_subcores=NS
    )

    @functools.partial(
        pl.kernel,
        mesh=mesh,
        out_type=jax.ShapeDtypeStruct((nw * n,), jnp.float32),
        scratch_types=[
            pltpu.VMEM((ec,), jnp.int32),
            pltpu.VMEM((n,), jnp.float32),
        ],
        compiler_params=pltpu.CompilerParams(needs_layout_passes=False),
    )
    def k(col_hbm, deg_hbm, colv, degv):
        wid = lax.axis_index("s") * NC + lax.axis_index("c")
        pltpu.sync_copy(col_hbm.at[pl.ds(wid * ec, ec)], colv)

        def zbody(i, carry):
            degv[pl.ds(i * L, L)] = jnp.zeros((L,), jnp.float32)
            return carry

        lax.fori_loop(0, n // L, zbody, 0, unroll=4)

        ones = jnp.ones((L,), jnp.float32)

        def cbody(i, carry):
            idx = colv[pl.ds(i * L, L)]
            plsc.addupdate_scatter(degv, [idx], ones)
            return carry

        lax.fori_loop(0, ec // L, cbody, 0, unroll=4)
        pltpu.sync_copy(degv, deg_hbm.at[pl.ds(wid * n, n)])

    return k(col).reshape(nw, n)


def _sc_aggregate(u, row, col, n):
    e = row.shape[0]
    c_dim = u.shape[1]
    ec = e // (NC * NS)
    K = 80
    nchunk = ec // K
    rpt = 640
    npad = rpt * NS
    zr = 128
    last = n - rpt * (NS - 1)
    mesh = plsc.VectorSubcoreMesh(
        core_axis_name="c", subcore_axis_name="s", num_cores=NC, num_subcores=NS
    )

    @functools.partial(
        pl.kernel,
        mesh=mesh,
        out_type=jax.ShapeDtypeStruct((NC, n, c_dim), jnp.float32),
        scratch_types=[
            pltpu.VMEM((2, K), jnp.int32),
            pltpu.VMEM((2, K), jnp.int32),
            pltpu.VMEM((2, K, c_dim), jnp.float32),
            pltpu.VMEM((zr, c_dim), jnp.float32),
            pltpu.VMEM_SHARED((npad, c_dim), jnp.float32),
            pltpu.SemaphoreType.DMA,
            pltpu.SemaphoreType.DMA,
        ],
        compiler_params=pltpu.CompilerParams(needs_layout_passes=False),
    )
    def k(u_hbm, row_hbm, col_hbm, out_hbm, rowv, colv, gbuf, zbuf, acc, sem0, sem1):
        cid = lax.axis_index("c")
        sid = lax.axis_index("s")

        nvec = c_dim // L

        def zbody(i, carry):
            r = i // nvec
            col0 = (i % nvec) * L
            zbuf[r, pl.ds(col0, L)] = jnp.zeros((L,), jnp.float32)
            return carry

        lax.fori_loop(0, zr * nvec, zbody, 0, unroll=4)
        for m in range(rpt // zr):
            pltpu.sync_copy(zbuf, acc.at[pl.ds(sid * rpt + m * zr, zr)])
        plsc.subcore_barrier()

        base = (cid * NS + sid) * ec
        sems = (sem0, sem1)

        def stage(j, slot):
            off = base + j * K
            pltpu.sync_copy(row_hbm.at[pl.ds(off, K)], rowv.at[slot])
            pltpu.sync_copy(col_hbm.at[pl.ds(off, K)], colv.at[slot])
            return pltpu.async_copy(u_hbm.at[rowv.at[slot]], gbuf.at[slot], sems[slot])

        def drain(slot, cp):
            cp.wait()
            pltpu.sync_copy(gbuf.at[slot], acc.at[colv.at[slot]], add=True)

        def pair(g, carry):
            cp0 = stage(2 * g, 0)
            cp1 = stage(2 * g + 1, 1)
            drain(0, cp0)
            drain(1, cp1)
            return carry

        lax.fori_loop(0, nchunk // 2, pair, 0)
        if nchunk % 2:
            drain(0, stage(nchunk - 1, 0))
        plsc.subcore_barrier()

        @pl.when(sid < NS - 1)
        def _():
            pltpu.sync_copy(
                acc.at[pl.ds(sid * rpt, rpt)], out_hbm.at[cid, pl.ds(sid * rpt, rpt)]
            )

        @pl.when(sid == NS - 1)
        def _():
            pltpu.sync_copy(
                acc.at[pl.ds((NS - 1) * rpt, last)],
                out_hbm.at[cid, pl.ds((NS - 1) * rpt, last)],
            )

    return k(u, row, col)


def _tc_prep(deg_part, x):
    n, c_dim = x.shape

    nw = deg_part.shape[0]

    def k(dp_ref, x_ref, u_ref, dinv_ref):
        ones = jnp.ones((nw, 1), jnp.float32)
        deg = 1.0 + lax.dot_general(
            dp_ref[...], ones, (((0,), (0,)), ((), ())),
            preferred_element_type=jnp.float32,
        )
        dinv = lax.rsqrt(deg)
        u_ref[...] = x_ref[...] * dinv
        dinv_ref[...] = dinv

    return pl.pallas_call(
        k,
        out_shape=[
            jax.ShapeDtypeStruct((n, c_dim), jnp.float32),
            jax.ShapeDtypeStruct((n, 1), jnp.float32),
        ],
    )(deg_part, x)


def _tc_linear(agg, x, dinv, W, b):
    n, c_dim = x.shape
    blk = 1000
    grid = n // blk

    def k(agg_ref, x_ref, dinv_ref, w_ref, b_ref, y_ref, st_ref):
        i = pl.program_id(0)
        dinv = dinv_ref[...]
        pre = agg_ref[0] * dinv + agg_ref[1] * dinv + x_ref[...] * (dinv * dinv)
        y = jnp.dot(pre, w_ref[...], preferred_element_type=jnp.float32) + b_ref[...]
        y_ref[...] = y

        @pl.when(i == 0)
        def _():
            st_ref[...] = jnp.zeros_like(st_ref)

        st_ref[0:1, :] += jnp.sum(y, axis=0, keepdims=True)
        st_ref[1:2, :] += jnp.sum(y * y, axis=0, keepdims=True)

    return pl.pallas_call(
        k,
        grid=(grid,),
        in_specs=[
            pl.BlockSpec((2, blk, c_dim), lambda i: (0, i, 0)),
            pl.BlockSpec((blk, c_dim), lambda i: (i, 0)),
            pl.BlockSpec((blk, 1), lambda i: (i, 0)),
            pl.BlockSpec((c_dim, c_dim), lambda i: (0, 0)),
            pl.BlockSpec((1, c_dim), lambda i: (0, 0)),
        ],
        out_specs=[
            pl.BlockSpec((blk, c_dim), lambda i: (i, 0)),
            pl.BlockSpec((2, c_dim), lambda i: (0, 0)),
        ],
        out_shape=[
            jax.ShapeDtypeStruct((n, c_dim), jnp.float32),
            jax.ShapeDtypeStruct((2, c_dim), jnp.float32),
        ],
    )(agg, x, dinv, W, b)


def _tc_bn(y, st, gamma, beta, n):
    _, c_dim = y.shape
    blk = 1000
    grid = n // blk
    inv_n = 1.0 / n

    def k(y_ref, st_ref, g_ref, bt_ref, o_ref):
        mean = st_ref[0:1, :] * inv_n
        var = st_ref[1:2, :] * inv_n - mean * mean
        scale = lax.rsqrt(var + EPS) * g_ref[...]
        o_ref[...] = jnp.maximum((y_ref[...] - mean) * scale + bt_ref[...], 0.0)

    return pl.pallas_call(
        k,
        grid=(grid,),
        in_specs=[
            pl.BlockSpec((blk, c_dim), lambda i: (i, 0)),
            pl.BlockSpec((2, c_dim), lambda i: (0, 0)),
            pl.BlockSpec((1, c_dim), lambda i: (0, 0)),
            pl.BlockSpec((1, c_dim), lambda i: (0, 0)),
        ],
        out_specs=pl.BlockSpec((blk, c_dim), lambda i: (i, 0)),
        out_shape=jax.ShapeDtypeStruct((n, c_dim), jnp.float32),
    )(y, st, gamma, beta)


def kernel(x, edge_index, W, b, gamma, beta):
    n = x.shape[0]
    row = edge_index[0]
    col = edge_index[1]
    deg_part = _sc_degree(col, n)
    u, dinv = _tc_prep(deg_part, x)
    agg = _sc_aggregate(u, row, col, n)
    y, st = _tc_linear(agg, x, dinv, W, b.reshape(1, -1))
    return _tc_bn(y, st, gamma.reshape(1, -1), beta.reshape(1, -1), n)


# speedup vs baseline: 36.9133x; 1.3784x over previous
import functools

import jax
import jax.numpy as jnp
from jax import lax
from jax.experimental import pallas as pl
from jax.experimental.pallas import tpu as pltpu
from jax.experimental.pallas import tpu_sc as plsc

NC = 2
NS = 16
L = 16
EPS = 1e-5


def _sc_degree(col, n):
    e = col.shape[0]
    nw = NC * NS
    ec = e // nw
    mesh = plsc.VectorSubcoreMesh(
        core_axis_name="c", subcore_axis_name="s", num_cores=NC, num_subcores=NS
    )

    @functools.partial(
        pl.kernel,
        mesh=mesh,
        out_type=jax.ShapeDtypeStruct((nw * n,), jnp.float32),
        scratch_types=[
            pltpu.VMEM((ec,), jnp.int32),
            pltpu.VMEM((n,), jnp.float32),
        ],
        compiler_params=pltpu.CompilerParams(needs_layout_passes=False),
    )
    def k(col_hbm, deg_hbm, colv, degv):
        wid = lax.axis_index("s") * NC + lax.axis_index("c")
        pltpu.sync_copy(col_hbm.at[pl.ds(wid * ec, ec)], colv)

        def zbody(i, carry):
            degv[pl.ds(i * L, L)] = jnp.zeros((L,), jnp.float32)
            return carry

        lax.fori_loop(0, n // L, zbody, 0, unroll=4)

        ones = jnp.ones((L,), jnp.float32)

        def cbody(i, carry):
            idx = colv[pl.ds(i * L, L)]
            plsc.addupdate_scatter(degv, [idx], ones)
            return carry

        lax.fori_loop(0, ec // L, cbody, 0, unroll=4)
        pltpu.sync_copy(degv, deg_hbm.at[pl.ds(wid * n, n)])

    return k(col).reshape(nw, n)


def _sc_aggregate(u, row, col, n):
    e = row.shape[0]
    c_dim = u.shape[1]
    ec = e // (NC * NS)
    K = 80
    nchunk = ec // K
    NBUF = 2
    ngrp = nchunk // NBUF
    tail = nchunk % NBUF
    rpt = 632
    last = n - rpt * (NS - 1)
    mesh = plsc.VectorSubcoreMesh(
        core_axis_name="c", subcore_axis_name="s", num_cores=NC, num_subcores=NS
    )

    row2 = row.reshape(NC * NS, ec)
    col3 = col.reshape(NC * NS, nchunk, K)

    @functools.partial(
        pl.kernel,
        mesh=mesh,
        out_type=jax.ShapeDtypeStruct((NC, n, c_dim), jnp.float32),
        scratch_types=[
            pltpu.VMEM((ec,), jnp.int32),
            pltpu.VMEM((nchunk, K), jnp.int32),
            pltpu.VMEM((NBUF, K, c_dim), jnp.float32),
            pltpu.VMEM_SHARED((n, c_dim), jnp.float32),
            pltpu.SemaphoreType.DMA,
            pltpu.SemaphoreType.DMA,
        ],
        compiler_params=pltpu.CompilerParams(needs_layout_passes=False),
    )
    def k(u_hbm, row_hbm, col_hbm, out_hbm, rowv, colv, gbuf, acc, gsem, ssem):
        cid = lax.axis_index("c")
        sid = lax.axis_index("s")
        wid = cid * NS + sid

        rcp = pltpu.async_copy(row_hbm.at[wid], rowv, gsem)
        ccp = pltpu.async_copy(col_hbm.at[wid], colv, ssem)

        nvec = c_dim // L

        def zbody(i, carry):
            r = i // nvec
            col0 = (i % nvec) * L
            gbuf[0, r, pl.ds(col0, L)] = jnp.zeros((L,), jnp.float32)
            return carry

        lax.fori_loop(0, K * nvec, zbody, 0, unroll=4)
        base_r = sid * rpt

        @pl.when(sid < NS - 1)
        def _():
            for m in range(rpt // K):
                pltpu.sync_copy(gbuf.at[0], acc.at[pl.ds(base_r + m * K, K)])
            rem = rpt % K
            pltpu.sync_copy(
                gbuf.at[0, pl.ds(0, rem)],
                acc.at[pl.ds(base_r + (rpt // K) * K, rem)],
            )

        @pl.when(sid == NS - 1)
        def _():
            for m in range(last // K):
                pltpu.sync_copy(gbuf.at[0], acc.at[pl.ds(base_r + m * K, K)])
            rem = last % K
            pltpu.sync_copy(
                gbuf.at[0, pl.ds(0, rem)],
                acc.at[pl.ds(base_r + (last // K) * K, rem)],
            )

        rcp.wait()
        ccp.wait()
        plsc.subcore_barrier()

        def fire(j, b):
            return pltpu.async_copy(
                u_hbm.at[rowv.at[pl.ds(j * K, K)]], gbuf.at[b], gsem
            )

        def grp(g, carry):
            j0 = g * NBUF
            cps = [fire(j0 + b, b) for b in range(NBUF)]
            scps = []
            for b in range(NBUF):
                cps[b].wait()
                scps.append(
                    pltpu.async_copy(gbuf.at[b], acc.at[colv.at[j0 + b]], ssem, add=True)
                )
            for s in scps:
                s.wait()
            return carry

        lax.fori_loop(0, ngrp, grp, 0)
        if tail:
            tcps = [fire(ngrp * NBUF + b, b) for b in range(tail)]
            tscps = []
            for b in range(tail):
                tcps[b].wait()
                tscps.append(
                    pltpu.async_copy(
                        gbuf.at[b], acc.at[colv.at[ngrp * NBUF + b]], ssem, add=True
                    )
                )
            for s in tscps:
                s.wait()
        plsc.subcore_barrier()

        @pl.when(sid < NS - 1)
        def _():
            pltpu.sync_copy(
                acc.at[pl.ds(sid * rpt, rpt)], out_hbm.at[cid, pl.ds(sid * rpt, rpt)]
            )

        @pl.when(sid == NS - 1)
        def _():
            pltpu.sync_copy(
                acc.at[pl.ds((NS - 1) * rpt, last)],
                out_hbm.at[cid, pl.ds((NS - 1) * rpt, last)],
            )

    return k(u, row2, col3)


def _tc_prep(deg_part, x):
    n, c_dim = x.shape

    nw = deg_part.shape[0]

    def k(dp_ref, x_ref, u_ref, dinv_ref):
        ones = jnp.ones((nw, 1), jnp.float32)
        deg = 1.0 + lax.dot_general(
            dp_ref[...], ones, (((0,), (0,)), ((), ())),
            preferred_element_type=jnp.float32,
        )
        dinv = lax.rsqrt(deg)
        u_ref[...] = x_ref[...] * dinv
        dinv_ref[...] = dinv

    return pl.pallas_call(
        k,
        out_shape=[
            jax.ShapeDtypeStruct((n, c_dim), jnp.float32),
            jax.ShapeDtypeStruct((n, 1), jnp.float32),
        ],
    )(deg_part, x)


def _tc_linear(agg, x, dinv, W, b):
    n, c_dim = x.shape
    blk = 1000
    grid = n // blk

    def k(agg_ref, x_ref, dinv_ref, w_ref, b_ref, y_ref, st_ref):
        i = pl.program_id(0)
        dinv = dinv_ref[...]
        pre = agg_ref[0] * dinv + agg_ref[1] * dinv + x_ref[...] * (dinv * dinv)
        y = jnp.dot(pre, w_ref[...], preferred_element_type=jnp.float32) + b_ref[...]
        y_ref[...] = y

        @pl.when(i == 0)
        def _():
            st_ref[...] = jnp.zeros_like(st_ref)

        st_ref[0:1, :] += jnp.sum(y, axis=0, keepdims=True)
        st_ref[1:2, :] += jnp.sum(y * y, axis=0, keepdims=True)

    return pl.pallas_call(
        k,
        grid=(grid,),
        in_specs=[
            pl.BlockSpec((2, blk, c_dim), lambda i: (0, i, 0)),
            pl.BlockSpec((blk, c_dim), lambda i: (i, 0)),
            pl.BlockSpec((blk, 1), lambda i: (i, 0)),
            pl.BlockSpec((c_dim, c_dim), lambda i: (0, 0)),
            pl.BlockSpec((1, c_dim), lambda i: (0, 0)),
        ],
        out_specs=[
            pl.BlockSpec((blk, c_dim), lambda i: (i, 0)),
            pl.BlockSpec((2, c_dim), lambda i: (0, 0)),
        ],
        out_shape=[
            jax.ShapeDtypeStruct((n, c_dim), jnp.float32),
            jax.ShapeDtypeStruct((2, c_dim), jnp.float32),
        ],
    )(agg, x, dinv, W, b)


def _tc_bn(y, st, gamma, beta, n):
    _, c_dim = y.shape
    blk = 1000
    grid = n // blk
    inv_n = 1.0 / n

    def k(y_ref, st_ref, g_ref, bt_ref, o_ref):
        mean = st_ref[0:1, :] * inv_n
        var = st_ref[1:2, :] * inv_n - mean * mean
        scale = lax.rsqrt(var + EPS) * g_ref[...]
        o_ref[...] = jnp.maximum((y_ref[...] - mean) * scale + bt_ref[...], 0.0)

    return pl.pallas_call(
        k,
        grid=(grid,),
        in_specs=[
            pl.BlockSpec((blk, c_dim), lambda i: (i, 0)),
            pl.BlockSpec((2, c_dim), lambda i: (0, 0)),
            pl.BlockSpec((1, c_dim), lambda i: (0, 0)),
            pl.BlockSpec((1, c_dim), lambda i: (0, 0)),
        ],
        out_specs=pl.BlockSpec((blk, c_dim), lambda i: (i, 0)),
        out_shape=jax.ShapeDtypeStruct((n, c_dim), jnp.float32),
    )(y, st, gamma, beta)


def kernel(x, edge_index, W, b, gamma, beta):
    n = x.shape[0]
    row = edge_index[0]
    col = edge_index[1]
    deg_part = _sc_degree(col, n)
    u, dinv = _tc_prep(deg_part, x)
    agg = _sc_aggregate(u, row, col, n)
    y, st = _tc_linear(agg, x, dinv, W, b.reshape(1, -1))
    return _tc_bn(y, st, gamma.reshape(1, -1), beta.reshape(1, -1), n)


# speedup vs baseline: 38.1682x; 1.0340x over previous
import functools

import jax
import jax.numpy as jnp
from jax import lax
from jax.experimental import pallas as pl
from jax.experimental.pallas import tpu as pltpu
from jax.experimental.pallas import tpu_sc as plsc

NC = 2
NS = 16
L = 16
EPS = 1e-5


def _sc_degree(col, n):
    e = col.shape[0]
    nw = NC * NS
    ec = e // nw
    mesh = plsc.VectorSubcoreMesh(
        core_axis_name="c", subcore_axis_name="s", num_cores=NC, num_subcores=NS
    )

    @functools.partial(
        pl.kernel,
        mesh=mesh,
        out_type=jax.ShapeDtypeStruct((nw * n,), jnp.float32),
        scratch_types=[
            pltpu.VMEM((ec,), jnp.int32),
            pltpu.VMEM((n,), jnp.float32),
        ],
        compiler_params=pltpu.CompilerParams(needs_layout_passes=False),
    )
    def k(col_hbm, deg_hbm, colv, degv):
        wid = lax.axis_index("s") * NC + lax.axis_index("c")
        pltpu.sync_copy(col_hbm.at[pl.ds(wid * ec, ec)], colv)

        def zbody(i, carry):
            degv[pl.ds(i * L, L)] = jnp.zeros((L,), jnp.float32)
            return carry

        lax.fori_loop(0, n // L, zbody, 0, unroll=4)

        ones = jnp.ones((L,), jnp.float32)

        def cbody(i, carry):
            idx = colv[pl.ds(i * L, L)]
            plsc.addupdate_scatter(degv, [idx], ones)
            return carry

        lax.fori_loop(0, ec // L, cbody, 0, unroll=4)
        pltpu.sync_copy(degv, deg_hbm.at[pl.ds(wid * n, n)])

    return k(col).reshape(nw, n)


def _sc_aggregate(u, row, col, n):
    e = row.shape[0]
    c_dim = u.shape[1]
    ec = e // (NC * NS)
    K = 80
    nchunk = ec // K
    NBUF = 4
    ngrp = nchunk // NBUF
    assert nchunk % NBUF == 1 and (ngrp - 1) % 2 == 0
    npair = (ngrp - 1) // 2
    rpt = 632
    last = n - rpt * (NS - 1)
    mesh = plsc.VectorSubcoreMesh(
        core_axis_name="c", subcore_axis_name="s", num_cores=NC, num_subcores=NS
    )

    nci = nchunk + 3
    idx3 = jnp.zeros((NC * NS, nci, 2, K), jnp.int32)
    idx3 = idx3.at[:, :nchunk].set(
        jnp.stack(
            [row.reshape(NC * NS, nchunk, K), col.reshape(NC * NS, nchunk, K)],
            axis=2,
        )
    )

    @functools.partial(
        pl.kernel,
        mesh=mesh,
        out_type=jax.ShapeDtypeStruct((NC, n, c_dim), jnp.float32),
        scratch_types=[
            pltpu.VMEM((2, NBUF, 2, K), jnp.int32),
            pltpu.VMEM((NBUF, K, c_dim), jnp.float32),
            pltpu.VMEM_SHARED((n, c_dim), jnp.float32),
            pltpu.SemaphoreType.DMA,
            pltpu.SemaphoreType.DMA,
            pltpu.SemaphoreType.DMA,
        ],
        compiler_params=pltpu.CompilerParams(needs_layout_passes=False),
    )
    def k(u_hbm, idx_hbm, out_hbm, idxv, gbuf, acc, gsem, ssem, isem):
        cid = lax.axis_index("c")
        sid = lax.axis_index("s")
        wid = cid * NS + sid

        pltpu.async_copy(idx_hbm.at[wid, pl.ds(0, NBUF)], idxv.at[0], isem)
        pltpu.async_copy(idx_hbm.at[wid, pl.ds(NBUF, NBUF)], idxv.at[1], isem)

        def wait_idx(p):
            pltpu.make_async_copy(
                idx_hbm.at[wid, pl.ds(0, NBUF)], idxv.at[p], isem
            ).wait()

        nvec = c_dim // L

        def zbody(i, carry):
            r = i // nvec
            col0 = (i % nvec) * L
            gbuf[0, r, pl.ds(col0, L)] = jnp.zeros((L,), jnp.float32)
            return carry

        lax.fori_loop(0, K * nvec, zbody, 0, unroll=4)
        base_r = sid * rpt

        @pl.when(sid < NS - 1)
        def _():
            for m in range(rpt // K):
                pltpu.sync_copy(gbuf.at[0], acc.at[pl.ds(base_r + m * K, K)])
            rem = rpt % K
            pltpu.sync_copy(
                gbuf.at[0, pl.ds(0, rem)],
                acc.at[pl.ds(base_r + (rpt // K) * K, rem)],
            )

        @pl.when(sid == NS - 1)
        def _():
            for m in range(last // K):
                pltpu.sync_copy(gbuf.at[0], acc.at[pl.ds(base_r + m * K, K)])
            rem = last % K
            pltpu.sync_copy(
                gbuf.at[0, pl.ds(0, rem)],
                acc.at[pl.ds(base_r + (last // K) * K, rem)],
            )

        plsc.subcore_barrier()

        def run_group(p, nb):
            cps = [
                pltpu.async_copy(u_hbm.at[idxv.at[p, b, 0]], gbuf.at[b], gsem)
                for b in range(nb)
            ]
            scps = []
            for b in range(nb):
                cps[b].wait()
                scps.append(
                    pltpu.async_copy(gbuf.at[b], acc.at[idxv.at[p, b, 1]], ssem, add=True)
                )
            for s in scps:
                s.wait()

        def pair(h, carry):
            for p in range(2):
                g = 2 * h + p
                wait_idx(p)
                run_group(p, NBUF)
                pltpu.async_copy(
                    idx_hbm.at[wid, pl.ds((g + 2) * NBUF, NBUF)], idxv.at[p], isem
                )
            return carry

        lax.fori_loop(0, npair, pair, 0)
        wait_idx(0)
        run_group(0, NBUF)
        wait_idx(1)
        run_group(1, 1)
        plsc.subcore_barrier()

        @pl.when(sid < NS - 1)
        def _():
            pltpu.sync_copy(
                acc.at[pl.ds(sid * rpt, rpt)], out_hbm.at[cid, pl.ds(sid * rpt, rpt)]
            )

        @pl.when(sid == NS - 1)
        def _():
            pltpu.sync_copy(
                acc.at[pl.ds((NS - 1) * rpt, last)],
                out_hbm.at[cid, pl.ds((NS - 1) * rpt, last)],
            )

    return k(u, idx3)


def _tc_prep(deg_part, x):
    n, c_dim = x.shape

    nw = deg_part.shape[0]

    def k(dp_ref, x_ref, u_ref, dinv_ref):
        ones = jnp.ones((nw, 1), jnp.float32)
        deg = 1.0 + lax.dot_general(
            dp_ref[...], ones, (((0,), (0,)), ((), ())),
            preferred_element_type=jnp.float32,
        )
        dinv = lax.rsqrt(deg)
        u_ref[...] = x_ref[...] * dinv
        dinv_ref[...] = dinv

    return pl.pallas_call(
        k,
        out_shape=[
            jax.ShapeDtypeStruct((n, c_dim), jnp.float32),
            jax.ShapeDtypeStruct((n, 1), jnp.float32),
        ],
    )(deg_part, x)


def _tc_linear(agg, x, dinv, W, b):
    n, c_dim = x.shape
    blk = 1000
    grid = n // blk

    def k(agg_ref, x_ref, dinv_ref, w_ref, b_ref, y_ref, st_ref):
        i = pl.program_id(0)
        dinv = dinv_ref[...]
        pre = agg_ref[0] * dinv + agg_ref[1] * dinv + x_ref[...] * (dinv * dinv)
        y = jnp.dot(pre, w_ref[...], preferred_element_type=jnp.float32) + b_ref[...]
        y_ref[...] = y

        @pl.when(i == 0)
        def _():
            st_ref[...] = jnp.zeros_like(st_ref)

        st_ref[0:1, :] += jnp.sum(y, axis=0, keepdims=True)
        st_ref[1:2, :] += jnp.sum(y * y, axis=0, keepdims=True)

    return pl.pallas_call(
        k,
        grid=(grid,),
        in_specs=[
            pl.BlockSpec((2, blk, c_dim), lambda i: (0, i, 0)),
            pl.BlockSpec((blk, c_dim), lambda i: (i, 0)),
            pl.BlockSpec((blk, 1), lambda i: (i, 0)),
            pl.BlockSpec((c_dim, c_dim), lambda i: (0, 0)),
            pl.BlockSpec((1, c_dim), lambda i: (0, 0)),
        ],
        out_specs=[
            pl.BlockSpec((blk, c_dim), lambda i: (i, 0)),
            pl.BlockSpec((2, c_dim), lambda i: (0, 0)),
        ],
        out_shape=[
            jax.ShapeDtypeStruct((n, c_dim), jnp.float32),
            jax.ShapeDtypeStruct((2, c_dim), jnp.float32),
        ],
    )(agg, x, dinv, W, b)


def _tc_bn(y, st, gamma, beta, n):
    _, c_dim = y.shape
    blk = 1000
    grid = n // blk
    inv_n = 1.0 / n

    def k(y_ref, st_ref, g_ref, bt_ref, o_ref):
        mean = st_ref[0:1, :] * inv_n
        var = st_ref[1:2, :] * inv_n - mean * mean
        scale = lax.rsqrt(var + EPS) * g_ref[...]
        o_ref[...] = jnp.maximum((y_ref[...] - mean) * scale + bt_ref[...], 0.0)

    return pl.pallas_call(
        k,
        grid=(grid,),
        in_specs=[
            pl.BlockSpec((blk, c_dim), lambda i: (i, 0)),
            pl.BlockSpec((2, c_dim), lambda i: (0, 0)),
            pl.BlockSpec((1, c_dim), lambda i: (0, 0)),
            pl.BlockSpec((1, c_dim), lambda i: (0, 0)),
        ],
        out_specs=pl.BlockSpec((blk, c_dim), lambda i: (i, 0)),
        out_shape=jax.ShapeDtypeStruct((n, c_dim), jnp.float32),
    )(y, st, gamma, beta)


def kernel(x, edge_index, W, b, gamma, beta):
    n = x.shape[0]
    row = edge_index[0]
    col = edge_index[1]
    deg_part = _sc_degree(col, n)
    u, dinv = _tc_prep(deg_part, x)
    agg = _sc_aggregate(u, row, col, n)
    y, st = _tc_linear(agg, x, dinv, W, b.reshape(1, -1))
    return _tc_bn(y, st, gamma.reshape(1, -1), beta.reshape(1, -1), n)
